# baseline (speedup 1.0000x reference)
import functools

import jax
import jax.numpy as jnp
from jax import lax
from jax.experimental import pallas as pl
from jax.experimental.pallas import tpu as pltpu
from jax.experimental.pallas import tpu_sc as plsc

HID = 128
N_NODES = 10000
N_EDGES = 320000

NC, NS = 2, 16
NW = NC * NS
CHUNK = 80
NBUF = 5

EDGE_PER_W = N_EDGES // NW
EDGE_CHUNKS = EDGE_PER_W // CHUNK
NODE_WORKERS = 25
NODE_PER_W = N_NODES // NODE_WORKERS
NODE_CHUNKS = NODE_PER_W // CHUNK
LANES = 16

TC_ROWS = 2000
TC_GRID = N_EDGES // TC_ROWS


def _bcast_lane(vec, lane):
    idx = jnp.full((LANES,), lane, jnp.int32)
    dnums = lax.GatherDimensionNumbers(
        offset_dims=(), collapsed_slice_dims=(0,), start_index_map=(0,))
    return lax.gather(vec, idx[:, None], dnums, (1,),
                      mode=lax.GatherScatterMode.PROMISE_IN_BOUNDS)


def _tc_xemb(X, W_t1, W_t2):
    Xa = X[:, 0].reshape(TC_GRID, 1, TC_ROWS)
    Xb = X[:, 1].reshape(TC_GRID, 1, TC_ROWS)

    def body(xa_ref, xb_ref, w1_ref, w2_ref, out_ref):
        a = xa_ref[0, 0, :]
        b = xb_ref[0, 0, :]
        iot = lax.broadcasted_iota(jnp.int32, (TC_ROWS, 16), 1)
        oh_a = (a[:, None] == iot).astype(jnp.float32)
        oh_b = (b[:, None] == iot).astype(jnp.float32)
        out_ref[0] = (jnp.dot(oh_a, w1_ref[...], preferred_element_type=jnp.float32)
                      + jnp.dot(oh_b, w2_ref[...], preferred_element_type=jnp.float32))

    out = pl.pallas_call(
        body,
        grid=(TC_GRID,),
        in_specs=[
            pl.BlockSpec((1, 1, TC_ROWS), lambda i: (i, 0, 0)),
            pl.BlockSpec((1, 1, TC_ROWS), lambda i: (i, 0, 0)),
            pl.BlockSpec((16, HID), lambda i: (0, 0)),
            pl.BlockSpec((16, HID), lambda i: (0, 0)),
        ],
        out_specs=pl.BlockSpec((1, TC_ROWS, HID), lambda i: (i, 0, 0)),
        out_shape=jax.ShapeDtypeStruct((TC_GRID, TC_ROWS, HID), jnp.float32),
    )(Xa, Xb, W_t1, W_t2)
    return out.reshape(N_EDGES, HID)


def _sc_gather(x3, A3, W_x, W_ea):
    mesh = plsc.VectorSubcoreMesh(core_axis_name="c", subcore_axis_name="s")

    @functools.partial(
        pl.kernel,
        out_type=(
            jax.ShapeDtypeStruct((N_NODES, HID), jnp.float32),
            jax.ShapeDtypeStruct((N_EDGES, HID), jnp.float32),
        ),
        mesh=mesh,
        compiler_params=pltpu.CompilerParams(needs_layout_passes=False),
        cost_estimate=pl.CostEstimate(
            flops=0, transcendentals=0,
            bytes_accessed=(N_NODES + N_EDGES) * HID * 4),
        scratch_types=[
            pltpu.VMEM((EDGE_CHUNKS, CHUNK), jnp.int32),
            pltpu.VMEM((32, HID), jnp.float32),
            pltpu.VMEM((16, HID), jnp.float32),
            pltpu.VMEM((NBUF, CHUNK, HID), jnp.float32),
            pltpu.SemaphoreType.DMA((NBUF,)),
        ],
    )
    def k(x_hbm, A_hbm, Wx_hbm, Wea_hbm,
          out_x, out_A, idx_v, wx_v, wea_v, rows_v, wsems):
        wid = lax.axis_index("s") * NC + lax.axis_index("c")
        ebase = wid * EDGE_PER_W
        lanes = lax.iota(jnp.int32, LANES)

        pltpu.sync_copy(Wx_hbm, wx_v)
        pltpu.sync_copy(Wea_hbm, wea_v)

        def gen_phase(tab_v, out_hbm, n_chunks, base):
            def write(i, b):
                return pltpu.make_async_copy(
                    rows_v.at[b], out_hbm.at[pl.ds(base + i * CHUNK, CHUNK)],
                    wsems.at[b])

            def chunk(i, carry):
                b = lax.rem(i, NBUF)

                @pl.when(i >= NBUF)
                def _():
                    write(i - NBUF, b).wait()

                for g in range(CHUNK // LANES):
                    va = idx_v[i, pl.ds(g * LANES, LANES)]
                    for r in range(LANES):
                        row = g * LANES + r
                        bc = _bcast_lane(va, r)
                        vals = [plsc.load_gather(tab_v, [bc, lanes + (c * LANES)])
                                for c in range(HID // LANES)]
                        for c in range(HID // LANES):
                            rows_v[b, row, pl.ds(c * LANES, LANES)] = vals[c]

                write(i, b).start()
                return carry

            lax.fori_loop(0, n_chunks, chunk, 0)
            for t in range(min(NBUF, n_chunks)):
                i = n_chunks - min(NBUF, n_chunks) + t
                write(i, i % NBUF).wait()

        pltpu.sync_copy(A_hbm.at[wid], idx_v)
        gen_phase(wea_v, out_A, EDGE_CHUNKS, ebase)

        @pl.when(wid < NODE_WORKERS)
        def _node_phase():
            pltpu.sync_copy(x_hbm.at[wid], idx_v.at[pl.ds(0, NODE_CHUNKS)])
            gen_phase(wx_v, out_x, NODE_CHUNKS, wid * NODE_PER_W)

    return k(x3, A3, W_x, W_ea)


def kernel(x, A, X, W_x, W_ea, W_t1, W_t2):
    A3 = A.reshape(NW, EDGE_CHUNKS, CHUNK)
    x3 = x.reshape(NODE_WORKERS, NODE_CHUNKS, CHUNK)
    x_emb, A_emb = _sc_gather(x3, A3, W_x, W_ea)
    X_emb = _tc_xemb(X, W_t1, W_t2)
    return (x_emb, A_emb, X_emb)


# speedup vs baseline: 1.0033x; 1.0033x over previous
import functools

import jax
import jax.numpy as jnp
from jax import lax
from jax.experimental import pallas as pl
from jax.experimental.pallas import tpu as pltpu
from jax.experimental.pallas import tpu_sc as plsc

HID = 128
N_NODES = 10000
N_EDGES = 320000

NC, NS = 2, 16
NW = NC * NS
CHUNK = 80
NBUF = 5

EDGE_PER_W = N_EDGES // NW
EDGE_CHUNKS = EDGE_PER_W // CHUNK
NODE_WORKERS = 25
NODE_PER_W = N_NODES // NODE_WORKERS
NODE_CHUNKS = NODE_PER_W // CHUNK
LANES = 16

TC_ROWS = 2000
TC_GRID = N_EDGES // TC_ROWS


def _bcast_lane(vec, lane):
    idx = jnp.full((LANES,), lane, jnp.int32)
    dnums = lax.GatherDimensionNumbers(
        offset_dims=(), collapsed_slice_dims=(0,), start_index_map=(0,))
    return lax.gather(vec, idx[:, None], dnums, (1,),
                      mode=lax.GatherScatterMode.PROMISE_IN_BOUNDS)


def _tc_xemb(X, W_t1, W_t2):
    Xa = X[:, 0].reshape(TC_GRID, 1, TC_ROWS)
    Xb = X[:, 1].reshape(TC_GRID, 1, TC_ROWS)

    def body(xa_ref, xb_ref, w1_ref, w2_ref, out_ref):
        a = xa_ref[0, 0, :]
        b = xb_ref[0, 0, :]
        iot = lax.broadcasted_iota(jnp.int32, (TC_ROWS, 16), 1)
        oh_a = (a[:, None] == iot).astype(jnp.float32)
        oh_b = (b[:, None] == iot).astype(jnp.float32)
        out_ref[0] = (jnp.dot(oh_a, w1_ref[...], preferred_element_type=jnp.float32)
                      + jnp.dot(oh_b, w2_ref[...], preferred_element_type=jnp.float32))

    out = pl.pallas_call(
        body,
        grid=(TC_GRID,),
        cost_estimate=pl.CostEstimate(
            flops=2 * 2 * N_EDGES * 16 * HID, transcendentals=0,
            bytes_accessed=N_EDGES * HID * 4),
        in_specs=[
            pl.BlockSpec((1, 1, TC_ROWS), lambda i: (i, 0, 0)),
            pl.BlockSpec((1, 1, TC_ROWS), lambda i: (i, 0, 0)),
            pl.BlockSpec((16, HID), lambda i: (0, 0)),
            pl.BlockSpec((16, HID), lambda i: (0, 0)),
        ],
        out_specs=pl.BlockSpec((1, TC_ROWS, HID), lambda i: (i, 0, 0)),
        out_shape=jax.ShapeDtypeStruct((TC_GRID, TC_ROWS, HID), jnp.float32),
    )(Xa, Xb, W_t1, W_t2)
    return out.reshape(N_EDGES, HID)


def _sc_gather(x3, A3, W_x, W_ea):
    mesh = plsc.VectorSubcoreMesh(core_axis_name="c", subcore_axis_name="s")

    @functools.partial(
        pl.kernel,
        out_type=(
            jax.ShapeDtypeStruct((N_NODES, HID), jnp.float32),
            jax.ShapeDtypeStruct((N_EDGES, HID), jnp.float32),
        ),
        mesh=mesh,
        compiler_params=pltpu.CompilerParams(needs_layout_passes=False),
        cost_estimate=pl.CostEstimate(
            flops=0, transcendentals=0,
            bytes_accessed=(N_NODES + N_EDGES) * HID * 4),
        scratch_types=[
            pltpu.VMEM((EDGE_CHUNKS, CHUNK), jnp.int32),
            pltpu.VMEM((32, HID), jnp.float32),
            pltpu.VMEM((16, HID), jnp.float32),
            pltpu.VMEM((NBUF, CHUNK, HID), jnp.float32),
            pltpu.SemaphoreType.DMA((NBUF,)),
        ],
    )
    def k(x_hbm, A_hbm, Wx_hbm, Wea_hbm,
          out_x, out_A, idx_v, wx_v, wea_v, rows_v, wsems):
        wid = lax.axis_index("s") * NC + lax.axis_index("c")
        ebase = wid * EDGE_PER_W
        lanes = lax.iota(jnp.int32, LANES)

        pltpu.sync_copy(Wx_hbm, wx_v)
        pltpu.sync_copy(Wea_hbm, wea_v)

        def gen_phase(tab_v, out_hbm, n_chunks, base):
            def write(i, b):
                return pltpu.make_async_copy(
                    rows_v.at[b], out_hbm.at[pl.ds(base + i * CHUNK, CHUNK)],
                    wsems.at[b])

            def chunk(i, carry):
                b = lax.rem(i, NBUF)

                @pl.when(i >= NBUF)
                def _():
                    write(i - NBUF, b).wait()

                for g in range(CHUNK // LANES):
                    va = idx_v[i, pl.ds(g * LANES, LANES)]
                    for r in range(LANES):
                        row = g * LANES + r
                        bc = _bcast_lane(va, r)
                        vals = [plsc.load_gather(tab_v, [bc, lanes + (c * LANES)])
                                for c in range(HID // LANES)]
                        for c in range(HID // LANES):
                            rows_v[b, row, pl.ds(c * LANES, LANES)] = vals[c]

                write(i, b).start()
                return carry

            lax.fori_loop(0, n_chunks, chunk, 0)
            for t in range(min(NBUF, n_chunks)):
                i = n_chunks - min(NBUF, n_chunks) + t
                write(i, i % NBUF).wait()

        pltpu.sync_copy(A_hbm.at[wid], idx_v)
        gen_phase(wea_v, out_A, EDGE_CHUNKS, ebase)

        @pl.when(wid < NODE_WORKERS)
        def _node_phase():
            pltpu.sync_copy(x_hbm.at[wid], idx_v.at[pl.ds(0, NODE_CHUNKS)])
            gen_phase(wx_v, out_x, NODE_CHUNKS, wid * NODE_PER_W)

    return k(x3, A3, W_x, W_ea)


def kernel(x, A, X, W_x, W_ea, W_t1, W_t2):
    A3 = A.reshape(NW, EDGE_CHUNKS, CHUNK)
    x3 = x.reshape(NODE_WORKERS, NODE_CHUNKS, CHUNK)
    x_emb, A_emb = _sc_gather(x3, A3, W_x, W_ea)
    X_emb = _tc_xemb(X, W_t1, W_t2)
    return (x_emb, A_emb, X_emb)
